# baseline (speedup 1.0000x reference)
import jax
import jax.numpy as jnp
from jax.experimental import pallas as pl

_BLOCK = 4000


def _body(x_ref, w_ref, c_ref, o_ref):
    xb = x_ref[...]
    a = jnp.concatenate([xb, xb * xb], axis=1)
    m = jnp.dot(a, w_ref[...],
                preferred_element_type=jnp.float32)
    c = c_ref[...]
    c2 = jnp.sum(c * c, axis=1, keepdims=True)
    mt = m.T
    dist = jnp.maximum(mt + c2, 0.0)
    u = 1.0 / (1.0 + dist)
    s = jnp.sum(u, axis=0, keepdims=True)
    qt = u * (1.0 / s)
    o_ref[...] = qt.T


def kernel(x, centers):
    n, d = x.shape
    k = centers.shape[0]
    w = jnp.concatenate([-2.0 * centers.T, jnp.ones((d, k), jnp.float32)],
                        axis=0)
    grid = (n // _BLOCK,)
    return pl.pallas_call(
        _body,
        grid=grid,
        in_specs=[
            pl.BlockSpec((_BLOCK, d), lambda i: (i, 0)),
            pl.BlockSpec((2 * d, k), lambda i: (0, 0)),
            pl.BlockSpec((k, d), lambda i: (0, 0)),
        ],
        out_specs=pl.BlockSpec((_BLOCK, k), lambda i: (i, 0)),
        out_shape=jax.ShapeDtypeStruct((n, k), jnp.float32),
    )(x, w, centers)


# speedup vs baseline: 1.1476x; 1.1476x over previous
import jax
import jax.numpy as jnp
from jax.experimental import pallas as pl

_BLOCK = 10000


def _body(x_ref, w_ref, c_ref, o_ref):
    xb = x_ref[...]
    a = jnp.concatenate([xb, xb * xb], axis=1)
    m = jnp.dot(a, w_ref[...],
                preferred_element_type=jnp.float32)
    c = c_ref[...]
    c2 = jnp.sum(c * c, axis=1, keepdims=True)
    mt = m.T
    dist = jnp.maximum(mt + c2, 0.0)
    u = 1.0 / (1.0 + dist)
    s = jnp.sum(u, axis=0, keepdims=True)
    qt = u * (1.0 / s)
    o_ref[...] = qt.T


def kernel(x, centers):
    n, d = x.shape
    k = centers.shape[0]
    w = jnp.concatenate([-2.0 * centers.T, jnp.ones((d, k), jnp.float32)],
                        axis=0)
    grid = (n // _BLOCK,)
    return pl.pallas_call(
        _body,
        grid=grid,
        in_specs=[
            pl.BlockSpec((_BLOCK, d), lambda i: (i, 0)),
            pl.BlockSpec((2 * d, k), lambda i: (0, 0)),
            pl.BlockSpec((k, d), lambda i: (0, 0)),
        ],
        out_specs=pl.BlockSpec((_BLOCK, k), lambda i: (i, 0)),
        out_shape=jax.ShapeDtypeStruct((n, k), jnp.float32),
    )(x, w, centers)


# speedup vs baseline: 3.3312x; 2.9027x over previous
import jax
import jax.numpy as jnp
from jax.experimental import pallas as pl

_BLOCK = 12800


def _body(x_ref, c_ref, o_ref):
    xb = x_ref[...]
    xt = xb.T
    x2 = jnp.sum(xt * xt, axis=0, keepdims=True)
    c = c_ref[...]
    c2 = jnp.sum(c * c, axis=1, keepdims=True)
    m = jax.lax.dot_general(c, xt, (((1,), (0,)), ((), ())),
                            preferred_element_type=jnp.float32)
    dist = jnp.maximum(x2 + c2 - 2.0 * m, 0.0)
    u = 1.0 / (1.0 + dist)
    s = jnp.sum(u, axis=0, keepdims=True)
    o_ref[...] = u * (1.0 / s)


def kernel(x, centers):
    n, d = x.shape
    k = centers.shape[0]
    grid = (pl.cdiv(n, _BLOCK),)
    qt = pl.pallas_call(
        _body,
        grid=grid,
        in_specs=[
            pl.BlockSpec((_BLOCK, d), lambda i: (i, 0)),
            pl.BlockSpec((k, d), lambda i: (0, 0)),
        ],
        out_specs=pl.BlockSpec((k, _BLOCK), lambda i: (0, i)),
        out_shape=jax.ShapeDtypeStruct((k, n), jnp.float32),
    )(x, centers)
    return qt.T
